# Initial kernel scaffold; baseline (speedup 1.0000x reference)
import jax
import jax.numpy as jnp
from jax.experimental import pallas as pl


def kernel(x, edge_index, batch_video, batch_actor, act_cids, sact_cids, actor_cids, W1a, b1a, W1b, b1b, W2a, b2a, W2b, b2b, W3a, b3a, W3b, b3b, Wa1, ba1, Wa2, ba2, Ws1, bs1, Ws2, bs2, Wr1, br1, Wr2, br2, g1, be1, g2, be2, g3, be3):
    raise NotImplementedError("write your pallas kernel here")


# speedup vs baseline: 3.6902x; 3.6902x over previous
import functools

import jax
import jax.numpy as jnp
from jax import lax
from jax.experimental import pallas as pl
from jax.experimental.pallas import tpu as pltpu
from jax.experimental.pallas import tpu_sc as plsc

N = 10000
E = 320000
NUM_FEATS = 128
DIM = 256
NUM_ACT = 20
NUM_SACT = 91
NUM_ACTOR = 26
NUM_VIDEOS = 16
NUM_CHUNKS = 4
CHUNK = 2500
SEG = 9

NC = 2
NS = 16
SLAB = 640
E_PER_TILE = E // NS
K = 80
NCHUNK = E_PER_TILE // K


_MESH = dict(core_axis_name="c", subcore_axis_name="s", num_cores=NC,
             num_subcores=NS)


@functools.cache
def _make_sc_agg_fs():
    mesh = plsc.VectorSubcoreMesh(**_MESH)
    dh = DIM // 2

    @functools.partial(
        pl.kernel,
        out_type=jax.ShapeDtypeStruct((2 * N, dh), jnp.float32),
        mesh=mesh,
        scratch_types=[
            pltpu.VMEM_SHARED((N, dh), jnp.float32),
            pltpu.VMEM((K,), jnp.int32),
            pltpu.VMEM((K,), jnp.int32),
            pltpu.VMEM((K, dh), jnp.float32),
            pltpu.SemaphoreType.DMA,
        ],
    )
    def sc_agg(h_hbm, src2_hbm, dst_hbm, out_hbm, acc, gidx, didx, rows, sem):
        c = lax.axis_index("c")
        s = lax.axis_index("s")
        row0 = pl.multiple_of(jnp.minimum(s * SLAB, N - SLAB), 8)
        pltpu.sync_copy(
            h_hbm.at[pl.ds(pl.multiple_of(c * N + row0, 8), SLAB)],
            acc.at[pl.ds(row0, SLAB)],
        )
        plsc.subcore_barrier()

        e0 = s * E_PER_TILE

        def body(k, carry):
            off = e0 + k * K
            pltpu.sync_copy(src2_hbm.at[pl.ds(pl.multiple_of(c * E + off, 8), K)], gidx)
            pltpu.sync_copy(dst_hbm.at[pl.ds(pl.multiple_of(off, 8), K)], didx)
            pltpu.async_copy(h_hbm.at[gidx], rows, sem).wait()
            pltpu.sync_copy(rows, acc.at[didx], add=True)
            return carry

        lax.fori_loop(0, NCHUNK, body, 0, unroll=False)
        plsc.subcore_barrier()
        pltpu.sync_copy(
            acc.at[pl.ds(row0, SLAB)],
            out_hbm.at[pl.ds(pl.multiple_of(c * N + row0, 8), SLAB)],
        )

    return sc_agg


@functools.cache
def _make_sc_agg_es():
    mesh = plsc.VectorSubcoreMesh(**_MESH)
    dh = NUM_FEATS
    e_sc = E // NC
    e_tile = e_sc // NS
    nchunk = e_tile // K

    @functools.partial(
        pl.kernel,
        out_type=jax.ShapeDtypeStruct((2 * N, dh), jnp.float32),
        mesh=mesh,
        scratch_types=[
            pltpu.VMEM_SHARED((N, dh), jnp.float32),
            pltpu.VMEM((K,), jnp.int32),
            pltpu.VMEM((K,), jnp.int32),
            pltpu.VMEM((K, dh), jnp.float32),
            pltpu.SemaphoreType.DMA,
        ],
    )
    def sc_agg(x_hbm, src_hbm, dst_hbm, out_hbm, acc, gidx, didx, rows, sem):
        c = lax.axis_index("c")
        s = lax.axis_index("s")
        row0 = pl.multiple_of(jnp.minimum(s * SLAB, N - SLAB), 8)
        pltpu.sync_copy(x_hbm.at[pl.ds(row0, SLAB)], acc.at[pl.ds(row0, SLAB)])
        plsc.subcore_barrier()

        e0 = c * e_sc + s * e_tile

        def body(k, carry):
            off = pl.multiple_of(e0 + k * K, 8)
            pltpu.sync_copy(src_hbm.at[pl.ds(off, K)], gidx)
            pltpu.sync_copy(dst_hbm.at[pl.ds(off, K)], didx)
            pltpu.async_copy(x_hbm.at[gidx], rows, sem).wait()
            pltpu.sync_copy(rows, acc.at[didx], add=True)
            return carry

        lax.fori_loop(0, nchunk, body, 0, unroll=False)
        plsc.subcore_barrier()
        pltpu.sync_copy(
            acc.at[pl.ds(row0, SLAB)],
            out_hbm.at[pl.ds(pl.multiple_of(c * N + row0, 8), SLAB)],
        )

    return sc_agg


def _layer_body(dh, mode, h2_ref, *refs):
    if mode == "es":
        x_ref, wa_ref, ba_ref, wb_ref, bb_ref, g_ref, be_ref, out_ref = refs
    else:
        wa_ref, ba_ref, wb_ref, bb_ref, g_ref, be_ref, out_ref = refs
    lo = h2_ref[:N, :]
    hi = h2_ref[N:, :]
    if mode == "fs":
        t = (
            jnp.dot(lo, wa_ref[:dh, :], preferred_element_type=jnp.float32)
            + jnp.dot(hi, wa_ref[dh:, :], preferred_element_type=jnp.float32)
            + ba_ref[...]
        )
    else:
        z = lo + hi - x_ref[...]
        t = jnp.dot(z, wa_ref[...], preferred_element_type=jnp.float32) + ba_ref[...]
    t = jnp.maximum(t, 0.0)
    u = jnp.dot(t, wb_ref[...], preferred_element_type=jnp.float32) + bb_ref[...]
    u = jnp.maximum(u, 0.0)
    m = jnp.mean(u, axis=0, keepdims=True)
    v = jnp.mean(u * u, axis=0, keepdims=True) - m * m
    h = (u - m) * (g_ref[...] * jax.lax.rsqrt(v + 1e-5)) + be_ref[...]
    out_ref[:N, :] = h[:, : DIM // 2]
    out_ref[N:, :] = h[:, DIM // 2 :]


def _tc_layer(h2s, wa, ba, wb, bb, g, be, dh, mode, x=None):
    extra = (x,) if mode == "es" else ()
    return pl.pallas_call(
        functools.partial(_layer_body, dh, mode),
        out_shape=jax.ShapeDtypeStruct((2 * N, DIM // 2), jnp.float32),
    )(h2s, *extra, wa, ba.reshape(1, -1), wb, bb.reshape(1, -1),
      g.reshape(1, -1), be.reshape(1, -1))


def _log_softmax(x):
    x = x - jnp.max(x, axis=1, keepdims=True)
    return x - jnp.log(jnp.sum(jnp.exp(x), axis=1, keepdims=True))


def _ce_loss(logits, y_ref, nclass):
    ls = _log_softmax(logits)
    oh = (y_ref[...] == lax.broadcasted_iota(jnp.int32, (y_ref.shape[0], nclass), 1)
          ).astype(jnp.float32)
    return -jnp.sum(oh * ls) / y_ref.shape[0]


def _heads_body(h3_ref, bv_ref, ba_ref, act_ref, sact_ref, actor_ref,
                wa1_ref, ba1_ref, wa2_ref, ba2_ref,
                ws1_ref, bs1_ref, ws2_ref, bs2_ref,
                wr1_ref, br1_ref, wr2_ref, br2_ref,
                loss_ref, lact_ref, lsact_ref,
                logits_act_ref, logits_sact_ref, logits_role_ref):
    f32 = jnp.float32
    cdot = functools.partial(
        lax.dot_general,
        dimension_numbers=(((0,), (0,)), ((), ())),
        preferred_element_type=f32,
    )
    h_lo = h3_ref[:N, :]
    h_hi = h3_ref[N:, :]

    ohv = (bv_ref[...] == lax.broadcasted_iota(jnp.int32, (N, NUM_VIDEOS), 1)
           ).astype(f32)
    cnt = jnp.sum(ohv, axis=0, keepdims=True)
    inv = 1.0 / jnp.maximum(cnt, 1.0)
    pv = jnp.concatenate([cdot(ohv, h_lo), cdot(ohv, h_hi)], axis=1) * inv.T

    logits_act = (
        jnp.dot(jnp.maximum(jnp.dot(pv, wa1_ref[...], preferred_element_type=f32)
                            + ba1_ref[...], 0.0),
                wa2_ref[...], preferred_element_type=f32) + ba2_ref[...]
    )
    logits_sact = (
        jnp.dot(jnp.maximum(jnp.dot(pv, ws1_ref[...], preferred_element_type=f32)
                            + bs1_ref[...], 0.0),
                ws2_ref[...], preferred_element_type=f32) + bs2_ref[...]
    )

    embs = []
    for i in range(NUM_CHUNKS):
        sb = ba_ref[i * CHUNK:(i + 1) * CHUNK, :]
        oha = (sb == lax.broadcasted_iota(jnp.int32, (CHUNK, SEG), 1)).astype(f32)
        ccnt = jnp.sum(oha, axis=0, keepdims=True)
        cinv = 1.0 / jnp.maximum(ccnt, 1.0)
        s_lo = cdot(oha, h_lo[i * CHUNK:(i + 1) * CHUNK, :])
        s_hi = cdot(oha, h_hi[i * CHUNK:(i + 1) * CHUNK, :])
        emb = jnp.concatenate([s_lo, s_hi], axis=1) * cinv.T
        embs.append(emb[1:, :])
    emb_act = jnp.concatenate(embs, axis=0)

    logits_role = (
        jnp.dot(jnp.maximum(jnp.dot(emb_act, wr1_ref[...], preferred_element_type=f32)
                            + br1_ref[...], 0.0),
                wr2_ref[...], preferred_element_type=f32) + br2_ref[...]
    )

    loss_act = _ce_loss(logits_act, act_ref, NUM_ACT)
    loss_sact = _ce_loss(logits_sact, sact_ref, NUM_SACT)
    loss_role = _ce_loss(logits_role, actor_ref, NUM_ACTOR)

    loss_ref[...] = jnp.reshape(loss_role, (1, 1))
    lact_ref[...] = jnp.reshape(loss_act, (1, 1))
    lsact_ref[...] = jnp.reshape(loss_sact, (1, 1))
    logits_act_ref[...] = logits_act
    logits_sact_ref[...] = logits_sact
    logits_role_ref[...] = logits_role


def _tc_heads(h3s, bv, ba, act_cids, sact_cids, actor_cids,
              Wa1, ba1, Wa2, ba2, Ws1, bs1, Ws2, bs2, Wr1, br1, Wr2, br2):
    out_shape = (
        jax.ShapeDtypeStruct((1, 1), jnp.float32),
        jax.ShapeDtypeStruct((1, 1), jnp.float32),
        jax.ShapeDtypeStruct((1, 1), jnp.float32),
        jax.ShapeDtypeStruct((NUM_VIDEOS, NUM_ACT), jnp.float32),
        jax.ShapeDtypeStruct((NUM_VIDEOS, NUM_SACT), jnp.float32),
        jax.ShapeDtypeStruct((NUM_CHUNKS * (SEG - 1), NUM_ACTOR), jnp.float32),
    )
    return pl.pallas_call(_heads_body, out_shape=out_shape)(
        h3s,
        bv.reshape(N, 1), ba.reshape(N, 1),
        act_cids.reshape(-1, 1), sact_cids.reshape(-1, 1), actor_cids.reshape(-1, 1),
        Wa1, ba1.reshape(1, -1), Wa2, ba2.reshape(1, -1),
        Ws1, bs1.reshape(1, -1), Ws2, bs2.reshape(1, -1),
        Wr1, br1.reshape(1, -1), Wr2, br2.reshape(1, -1),
    )


def kernel(x, edge_index, batch_video, batch_actor, act_cids, sact_cids,
           actor_cids,
           W1a, b1a, W1b, b1b, W2a, b2a, W2b, b2b, W3a, b3a, W3b, b3b,
           Wa1, ba1, Wa2, ba2, Ws1, bs1, Ws2, bs2, Wr1, br1, Wr2, br2,
           g1, be1, g2, be2, g3, be3):
    src = edge_index[0]
    dst = edge_index[1]
    src2 = jnp.concatenate([src, src + N])

    h2 = _make_sc_agg_es()(x, src, dst)
    h = _tc_layer(h2, W1a, b1a, W1b, b1b, g1, be1, NUM_FEATS, "es", x=x)

    h2 = _make_sc_agg_fs()(h, src2, dst)
    h = _tc_layer(h2, W2a, b2a, W2b, b2b, g2, be2, DIM // 2, "fs")

    h2 = _make_sc_agg_fs()(h, src2, dst)
    h = _tc_layer(h2, W3a, b3a, W3b, b3b, g3, be3, DIM // 2, "fs")

    loss, lact, lsact, logits_act, logits_sact, logits_role = _tc_heads(
        h, batch_video, batch_actor, act_cids, sact_cids, actor_cids,
        Wa1, ba1, Wa2, ba2, Ws1, bs1, Ws2, bs2, Wr1, br1, Wr2, br2,
    )
    return (
        loss.reshape(()), lact.reshape(()), lsact.reshape(()),
        logits_act, logits_sact, logits_role,
    )


# speedup vs baseline: 9.3859x; 2.5435x over previous
import functools

import jax
import jax.numpy as jnp
from jax import lax
from jax.experimental import pallas as pl
from jax.experimental.pallas import tpu as pltpu
from jax.experimental.pallas import tpu_sc as plsc

N = 10000
E = 320000
NUM_FEATS = 128
DIM = 256
NUM_ACT = 20
NUM_SACT = 91
NUM_ACTOR = 26
NUM_VIDEOS = 16
NUM_CHUNKS = 4
CHUNK = 2500
SEG = 9

NC = 2
NS = 16
SLAB = 640
K = 80


NB = 4
BLK = 25


@functools.cache
def _make_sc_agg(nblk: int, init_stride: int):
    mesh = plsc.VectorSubcoreMesh(
        core_axis_name="c", subcore_axis_name="s", num_cores=NC, num_subcores=NS
    )
    dh = DIM // 2

    scratch = [
        pltpu.VMEM_SHARED((N, dh), jnp.float32),
        pltpu.VMEM((BLK, K), jnp.int32),
        pltpu.VMEM((BLK, K), jnp.int32),
    ]
    scratch += [pltpu.VMEM((K, dh), jnp.float32)] * NB
    scratch += [pltpu.SemaphoreType.DMA] * (2 * NB)

    @functools.partial(
        pl.kernel,
        out_type=jax.ShapeDtypeStruct((2 * N, dh), jnp.float32),
        mesh=mesh,
        scratch_types=scratch,
    )
    def sc_agg(tab_hbm, sidx_hbm, didx_hbm, out_hbm, acc, sidx, didx, *bufs):
        rows = bufs[:NB]
        gsem = bufs[NB:2 * NB]
        ssem = bufs[2 * NB:]
        c = lax.axis_index("c")
        s = lax.axis_index("s")
        w = c * NS + s
        row0 = pl.multiple_of(jnp.minimum(s * SLAB, N - SLAB), 8)
        pltpu.sync_copy(
            tab_hbm.at[pl.ds(pl.multiple_of(c * init_stride + row0, 8), SLAB)],
            acc.at[pl.ds(row0, SLAB)],
        )
        plsc.subcore_barrier()

        def fire_gather(k, b):
            pltpu.async_copy(tab_hbm.at[sidx.at[k]], rows[b], gsem[b])

        def wait_gather(k, b):
            pltpu.make_async_copy(tab_hbm.at[sidx.at[k]], rows[b], gsem[b]).wait()

        def fire_scatter(k, b):
            pltpu.async_copy(rows[b], acc.at[didx.at[k]], ssem[b], add=True)

        def wait_scatter(k, b):
            pltpu.make_async_copy(rows[b], acc.at[didx.at[k]], ssem[b]).wait()

        def step(k, b, do_wait=True, do_gather=True):
            tb = (b + 2) % NB
            if do_wait:
                wait_scatter(k - 2, tb)
            if do_gather:
                fire_gather(k + 2, tb)
            wait_gather(k, b)
            fire_scatter(k, b)

        k2_max = (BLK - 6) // NB

        def block_body(j, carry):
            pltpu.sync_copy(sidx_hbm.at[w, j], sidx)
            pltpu.sync_copy(didx_hbm.at[w, j], didx)
            fire_gather(0, 0)
            fire_gather(1, 1)
            for k in range(NB):
                step(k, k % NB, do_wait=k >= 2, do_gather=k + 2 < BLK)

            def round_body(k2, carry2):
                k0 = k2 * NB
                for b in range(NB):
                    step(k0 + b, b)
                return carry2

            lax.fori_loop(1, k2_max + 1, round_body, 0, unroll=False)
            for k in range((k2_max + 1) * NB, BLK):
                step(k, k % NB, do_gather=k + 2 < BLK)
            wait_scatter(BLK - 2, (BLK - 2) % NB)
            wait_scatter(BLK - 1, (BLK - 1) % NB)
            return carry

        lax.fori_loop(0, nblk, block_body, 0, unroll=False)

        plsc.subcore_barrier()
        pltpu.sync_copy(
            acc.at[pl.ds(row0, SLAB)],
            out_hbm.at[pl.ds(pl.multiple_of(c * N + row0, 8), SLAB)],
        )

    return sc_agg


def _layer_body(dh, mode, h2_ref, *refs):
    if mode == "es":
        x_ref, wa_ref, ba_ref, wb_ref, bb_ref, g_ref, be_ref, out_ref = refs
    else:
        wa_ref, ba_ref, wb_ref, bb_ref, g_ref, be_ref, out_ref = refs
    lo = h2_ref[:N, :]
    hi = h2_ref[N:, :]
    if mode == "fs":
        t = (
            jnp.dot(lo, wa_ref[:dh, :], preferred_element_type=jnp.float32)
            + jnp.dot(hi, wa_ref[dh:, :], preferred_element_type=jnp.float32)
            + ba_ref[...]
        )
    else:
        z = lo + hi - x_ref[...]
        t = jnp.dot(z, wa_ref[...], preferred_element_type=jnp.float32) + ba_ref[...]
    t = jnp.maximum(t, 0.0)
    u = jnp.dot(t, wb_ref[...], preferred_element_type=jnp.float32) + bb_ref[...]
    u = jnp.maximum(u, 0.0)
    m = jnp.mean(u, axis=0, keepdims=True)
    v = jnp.mean(u * u, axis=0, keepdims=True) - m * m
    h = (u - m) * (g_ref[...] * jax.lax.rsqrt(v + 1e-5)) + be_ref[...]
    out_ref[:N, :] = h[:, : DIM // 2]
    out_ref[N:, :] = h[:, DIM // 2 :]


def _tc_layer(h2s, wa, ba, wb, bb, g, be, dh, mode, x=None):
    extra = (x,) if mode == "es" else ()
    return pl.pallas_call(
        functools.partial(_layer_body, dh, mode),
        out_shape=jax.ShapeDtypeStruct((2 * N, DIM // 2), jnp.float32),
    )(h2s, *extra, wa, ba.reshape(1, -1), wb, bb.reshape(1, -1),
      g.reshape(1, -1), be.reshape(1, -1))


def _log_softmax(x):
    x = x - jnp.max(x, axis=1, keepdims=True)
    return x - jnp.log(jnp.sum(jnp.exp(x), axis=1, keepdims=True))


def _ce_loss(logits, y_ref, nclass):
    ls = _log_softmax(logits)
    oh = (y_ref[...] == lax.broadcasted_iota(jnp.int32, (y_ref.shape[0], nclass), 1)
          ).astype(jnp.float32)
    return -jnp.sum(oh * ls) / y_ref.shape[0]


def _heads_body(h3_ref, bv_ref, ba_ref, act_ref, sact_ref, actor_ref,
                wa1_ref, ba1_ref, wa2_ref, ba2_ref,
                ws1_ref, bs1_ref, ws2_ref, bs2_ref,
                wr1_ref, br1_ref, wr2_ref, br2_ref,
                loss_ref, lact_ref, lsact_ref,
                logits_act_ref, logits_sact_ref, logits_role_ref):
    f32 = jnp.float32
    cdot = functools.partial(
        lax.dot_general,
        dimension_numbers=(((0,), (0,)), ((), ())),
        preferred_element_type=f32,
    )
    h_lo = h3_ref[:N, :]
    h_hi = h3_ref[N:, :]

    ohv = (bv_ref[...] == lax.broadcasted_iota(jnp.int32, (N, NUM_VIDEOS), 1)
           ).astype(f32)
    cnt = jnp.sum(ohv, axis=0, keepdims=True)
    inv = 1.0 / jnp.maximum(cnt, 1.0)
    pv = jnp.concatenate([cdot(ohv, h_lo), cdot(ohv, h_hi)], axis=1) * inv.T

    logits_act = (
        jnp.dot(jnp.maximum(jnp.dot(pv, wa1_ref[...], preferred_element_type=f32)
                            + ba1_ref[...], 0.0),
                wa2_ref[...], preferred_element_type=f32) + ba2_ref[...]
    )
    logits_sact = (
        jnp.dot(jnp.maximum(jnp.dot(pv, ws1_ref[...], preferred_element_type=f32)
                            + bs1_ref[...], 0.0),
                ws2_ref[...], preferred_element_type=f32) + bs2_ref[...]
    )

    embs = []
    for i in range(NUM_CHUNKS):
        sb = ba_ref[i * CHUNK:(i + 1) * CHUNK, :]
        oha = (sb == lax.broadcasted_iota(jnp.int32, (CHUNK, SEG), 1)).astype(f32)
        ccnt = jnp.sum(oha, axis=0, keepdims=True)
        cinv = 1.0 / jnp.maximum(ccnt, 1.0)
        s_lo = cdot(oha, h_lo[i * CHUNK:(i + 1) * CHUNK, :])
        s_hi = cdot(oha, h_hi[i * CHUNK:(i + 1) * CHUNK, :])
        emb = jnp.concatenate([s_lo, s_hi], axis=1) * cinv.T
        embs.append(emb[1:, :])
    emb_act = jnp.concatenate(embs, axis=0)

    logits_role = (
        jnp.dot(jnp.maximum(jnp.dot(emb_act, wr1_ref[...], preferred_element_type=f32)
                            + br1_ref[...], 0.0),
                wr2_ref[...], preferred_element_type=f32) + br2_ref[...]
    )

    loss_act = _ce_loss(logits_act, act_ref, NUM_ACT)
    loss_sact = _ce_loss(logits_sact, sact_ref, NUM_SACT)
    loss_role = _ce_loss(logits_role, actor_ref, NUM_ACTOR)

    loss_ref[...] = jnp.reshape(loss_role, (1, 1))
    lact_ref[...] = jnp.reshape(loss_act, (1, 1))
    lsact_ref[...] = jnp.reshape(loss_sact, (1, 1))
    logits_act_ref[...] = logits_act
    logits_sact_ref[...] = logits_sact
    logits_role_ref[...] = logits_role


def _tc_heads(h3s, bv, ba, act_cids, sact_cids, actor_cids,
              Wa1, ba1, Wa2, ba2, Ws1, bs1, Ws2, bs2, Wr1, br1, Wr2, br2):
    out_shape = (
        jax.ShapeDtypeStruct((1, 1), jnp.float32),
        jax.ShapeDtypeStruct((1, 1), jnp.float32),
        jax.ShapeDtypeStruct((1, 1), jnp.float32),
        jax.ShapeDtypeStruct((NUM_VIDEOS, NUM_ACT), jnp.float32),
        jax.ShapeDtypeStruct((NUM_VIDEOS, NUM_SACT), jnp.float32),
        jax.ShapeDtypeStruct((NUM_CHUNKS * (SEG - 1), NUM_ACTOR), jnp.float32),
    )
    return pl.pallas_call(_heads_body, out_shape=out_shape)(
        h3s,
        bv.reshape(N, 1), ba.reshape(N, 1),
        act_cids.reshape(-1, 1), sact_cids.reshape(-1, 1), actor_cids.reshape(-1, 1),
        Wa1, ba1.reshape(1, -1), Wa2, ba2.reshape(1, -1),
        Ws1, bs1.reshape(1, -1), Ws2, bs2.reshape(1, -1),
        Wr1, br1.reshape(1, -1), Wr2, br2.reshape(1, -1),
    )


def kernel(x, edge_index, batch_video, batch_actor, act_cids, sact_cids,
           actor_cids,
           W1a, b1a, W1b, b1b, W2a, b2a, W2b, b2b, W3a, b3a, W3b, b3b,
           Wa1, ba1, Wa2, ba2, Ws1, bs1, Ws2, bs2, Wr1, br1, Wr2, br2,
           g1, be1, g2, be2, g3, be3):
    src = edge_index[0]
    dst = edge_index[1]
    nw = NC * NS

    nblk_es = E // nw // K // BLK
    src_es = src.reshape(nw, nblk_es, BLK, K)
    dst_es = dst.reshape(nw, nblk_es, BLK, K)

    nblk_fs = E // NS // K // BLK
    src_fs = jnp.concatenate([src, src + N]).reshape(nw, nblk_fs, BLK, K)
    dst_fs = jnp.concatenate([dst, dst]).reshape(nw, nblk_fs, BLK, K)

    h2 = _make_sc_agg(nblk_es, 0)(x, src_es, dst_es)
    h = _tc_layer(h2, W1a, b1a, W1b, b1b, g1, be1, NUM_FEATS, "es", x=x)

    h2 = _make_sc_agg(nblk_fs, N)(h, src_fs, dst_fs)
    h = _tc_layer(h2, W2a, b2a, W2b, b2b, g2, be2, DIM // 2, "fs")

    h2 = _make_sc_agg(nblk_fs, N)(h, src_fs, dst_fs)
    h = _tc_layer(h2, W3a, b3a, W3b, b3b, g3, be3, DIM // 2, "fs")

    loss, lact, lsact, logits_act, logits_sact, logits_role = _tc_heads(
        h, batch_video, batch_actor, act_cids, sact_cids, actor_cids,
        Wa1, ba1, Wa2, ba2, Ws1, bs1, Ws2, bs2, Wr1, br1, Wr2, br2,
    )
    return (
        loss.reshape(()), lact.reshape(()), lsact.reshape(()),
        logits_act, logits_sact, logits_role,
    )


# speedup vs baseline: 9.5579x; 1.0183x over previous
import functools

import jax
import jax.numpy as jnp
from jax import lax
from jax.experimental import pallas as pl
from jax.experimental.pallas import tpu as pltpu
from jax.experimental.pallas import tpu_sc as plsc

N = 10000
E = 320000
NUM_FEATS = 128
DIM = 256
NUM_ACT = 20
NUM_SACT = 91
NUM_ACTOR = 26
NUM_VIDEOS = 16
NUM_CHUNKS = 4
CHUNK = 2500
SEG = 9

NC = 2
NS = 16
SLAB = 640
K = 80


NB = 4
BLK = 25


@functools.cache
def _make_sc_agg(nblk: int, init_stride: int):
    mesh = plsc.VectorSubcoreMesh(
        core_axis_name="c", subcore_axis_name="s", num_cores=NC, num_subcores=NS
    )
    dh = DIM // 2

    scratch = [
        pltpu.VMEM_SHARED((N, dh), jnp.float32),
        pltpu.VMEM((BLK, K), jnp.int32),
        pltpu.VMEM((BLK, K), jnp.int32),
    ]
    scratch += [pltpu.VMEM((K, dh), jnp.float32)] * NB
    scratch += [pltpu.SemaphoreType.DMA] * (2 * NB)

    @functools.partial(
        pl.kernel,
        out_type=jax.ShapeDtypeStruct((2 * N, dh), jnp.float32),
        mesh=mesh,
        scratch_types=scratch,
    )
    def sc_agg(tab_hbm, sidx_hbm, didx_hbm, out_hbm, acc, sidx, didx, *bufs):
        rows = bufs[:NB]
        gsem = bufs[NB:2 * NB]
        ssem = bufs[2 * NB:]
        c = lax.axis_index("c")
        s = lax.axis_index("s")
        w = c * NS + s
        row0 = pl.multiple_of(jnp.minimum(s * SLAB, N - SLAB), 8)
        pltpu.sync_copy(
            tab_hbm.at[pl.ds(pl.multiple_of(c * init_stride + row0, 8), SLAB)],
            acc.at[pl.ds(row0, SLAB)],
        )
        plsc.subcore_barrier()

        def fire_gather(k, b):
            pltpu.async_copy(tab_hbm.at[sidx.at[k]], rows[b], gsem[b])

        def wait_gather(k, b):
            pltpu.make_async_copy(tab_hbm.at[sidx.at[k]], rows[b], gsem[b]).wait()

        def fire_scatter(k, b):
            pltpu.async_copy(rows[b], acc.at[didx.at[k]], ssem[b], add=True)

        def wait_scatter(k, b):
            pltpu.make_async_copy(rows[b], acc.at[didx.at[k]], ssem[b]).wait()

        def step(k, b, do_wait=True, do_gather=True):
            tb = (b + 2) % NB
            if do_wait:
                wait_scatter(k - 2, tb)
            if do_gather:
                fire_gather(k + 2, tb)
            wait_gather(k, b)
            fire_scatter(k, b)

        k2_max = (BLK - 6) // NB

        def block_body(j, carry):
            pltpu.sync_copy(sidx_hbm.at[w, j], sidx)
            pltpu.sync_copy(didx_hbm.at[w, j], didx)
            fire_gather(0, 0)
            fire_gather(1, 1)
            for k in range(NB):
                step(k, k % NB, do_wait=k >= 2, do_gather=k + 2 < BLK)

            def round_body(k2, carry2):
                k0 = k2 * NB
                for b in range(NB):
                    step(k0 + b, b)
                return carry2

            lax.fori_loop(1, k2_max + 1, round_body, 0, unroll=False)
            for k in range((k2_max + 1) * NB, BLK):
                step(k, k % NB, do_gather=k + 2 < BLK)
            wait_scatter(BLK - 2, (BLK - 2) % NB)
            wait_scatter(BLK - 1, (BLK - 1) % NB)
            return carry

        lax.fori_loop(0, nblk, block_body, 0, unroll=False)

        plsc.subcore_barrier()
        pltpu.sync_copy(
            acc.at[pl.ds(row0, SLAB)],
            out_hbm.at[pl.ds(pl.multiple_of(c * N + row0, 8), SLAB)],
        )

    return sc_agg


def _layer_body(dh, mode, h2_ref, *refs):
    if mode == "es":
        x_ref, wa_ref, ba_ref, wb_ref, bb_ref, g_ref, be_ref, out_ref = refs
    else:
        wa_ref, ba_ref, wb_ref, bb_ref, g_ref, be_ref, out_ref = refs
    lo = h2_ref[:N, :]
    hi = h2_ref[N:, :]
    if mode == "fs":
        t = (
            jnp.dot(lo, wa_ref[:dh, :], preferred_element_type=jnp.float32)
            + jnp.dot(hi, wa_ref[dh:, :], preferred_element_type=jnp.float32)
            + ba_ref[...]
        )
    else:
        z = lo + hi - x_ref[...]
        t = jnp.dot(z, wa_ref[...], preferred_element_type=jnp.float32) + ba_ref[...]
    t = jnp.maximum(t, 0.0)
    u = jnp.dot(t, wb_ref[...], preferred_element_type=jnp.float32) + bb_ref[...]
    u = jnp.maximum(u, 0.0)
    m = jnp.mean(u, axis=0, keepdims=True)
    v = jnp.mean(u * u, axis=0, keepdims=True) - m * m
    h = (u - m) * (g_ref[...] * jax.lax.rsqrt(v + 1e-5)) + be_ref[...]
    out_ref[:N, :] = h[:, : DIM // 2]
    out_ref[N:, :] = h[:, DIM // 2 :]


def _tc_layer(h2s, wa, ba, wb, bb, g, be, dh, mode, x=None):
    extra = (x,) if mode == "es" else ()
    return pl.pallas_call(
        functools.partial(_layer_body, dh, mode),
        out_shape=jax.ShapeDtypeStruct((2 * N, DIM // 2), jnp.float32),
    )(h2s, *extra, wa, ba.reshape(1, -1), wb, bb.reshape(1, -1),
      g.reshape(1, -1), be.reshape(1, -1))


def _log_softmax(x):
    x = x - jnp.max(x, axis=1, keepdims=True)
    return x - jnp.log(jnp.sum(jnp.exp(x), axis=1, keepdims=True))


def _ce_loss(logits, y_ref, nclass):
    ls = _log_softmax(logits)
    oh = (y_ref[...] == lax.broadcasted_iota(jnp.int32, (y_ref.shape[0], nclass), 1)
          ).astype(jnp.float32)
    return -jnp.sum(oh * ls) / y_ref.shape[0]


def _heads_body(h2_ref, wa_ref, ba_ref, wb_ref, bb_ref, g_ref, be_ref,
                bv_ref, ba2d_ref, act_ref, sact_ref, actor_ref,
                wa1_ref, ba1_ref, wa2_ref, ba2_ref,
                ws1_ref, bs1_ref, ws2_ref, bs2_ref,
                wr1_ref, br1_ref, wr2_ref, br2_ref,
                loss_ref, lact_ref, lsact_ref,
                logits_act_ref, logits_sact_ref, logits_role_ref):
    f32 = jnp.float32
    dh = DIM // 2
    cdot = functools.partial(
        lax.dot_general,
        dimension_numbers=(((0,), (0,)), ((), ())),
        preferred_element_type=f32,
    )
    t = (
        jnp.dot(h2_ref[:N, :], wa_ref[:dh, :], preferred_element_type=f32)
        + jnp.dot(h2_ref[N:, :], wa_ref[dh:, :], preferred_element_type=f32)
        + ba_ref[...]
    )
    t = jnp.maximum(t, 0.0)
    u = jnp.dot(t, wb_ref[...], preferred_element_type=f32) + bb_ref[...]
    u = jnp.maximum(u, 0.0)
    m = jnp.mean(u, axis=0, keepdims=True)
    v = jnp.mean(u * u, axis=0, keepdims=True) - m * m
    h = (u - m) * (g_ref[...] * jax.lax.rsqrt(v + 1e-5)) + be_ref[...]
    h_lo = h[:, :dh]
    h_hi = h[:, dh:]
    ba_ref = ba2d_ref

    ohv = (bv_ref[...] == lax.broadcasted_iota(jnp.int32, (N, NUM_VIDEOS), 1)
           ).astype(f32)
    cnt = jnp.sum(ohv, axis=0, keepdims=True)
    inv = 1.0 / jnp.maximum(cnt, 1.0)
    pv = jnp.concatenate([cdot(ohv, h_lo), cdot(ohv, h_hi)], axis=1) * inv.T

    logits_act = (
        jnp.dot(jnp.maximum(jnp.dot(pv, wa1_ref[...], preferred_element_type=f32)
                            + ba1_ref[...], 0.0),
                wa2_ref[...], preferred_element_type=f32) + ba2_ref[...]
    )
    logits_sact = (
        jnp.dot(jnp.maximum(jnp.dot(pv, ws1_ref[...], preferred_element_type=f32)
                            + bs1_ref[...], 0.0),
                ws2_ref[...], preferred_element_type=f32) + bs2_ref[...]
    )

    embs = []
    for i in range(NUM_CHUNKS):
        sb = ba_ref[i * CHUNK:(i + 1) * CHUNK, :]
        oha = (sb == lax.broadcasted_iota(jnp.int32, (CHUNK, SEG), 1)).astype(f32)
        ccnt = jnp.sum(oha, axis=0, keepdims=True)
        cinv = 1.0 / jnp.maximum(ccnt, 1.0)
        s_lo = cdot(oha, h_lo[i * CHUNK:(i + 1) * CHUNK, :])
        s_hi = cdot(oha, h_hi[i * CHUNK:(i + 1) * CHUNK, :])
        emb = jnp.concatenate([s_lo, s_hi], axis=1) * cinv.T
        embs.append(emb[1:, :])
    emb_act = jnp.concatenate(embs, axis=0)

    logits_role = (
        jnp.dot(jnp.maximum(jnp.dot(emb_act, wr1_ref[...], preferred_element_type=f32)
                            + br1_ref[...], 0.0),
                wr2_ref[...], preferred_element_type=f32) + br2_ref[...]
    )

    loss_act = _ce_loss(logits_act, act_ref, NUM_ACT)
    loss_sact = _ce_loss(logits_sact, sact_ref, NUM_SACT)
    loss_role = _ce_loss(logits_role, actor_ref, NUM_ACTOR)

    loss_ref[...] = jnp.reshape(loss_role, (1, 1))
    lact_ref[...] = jnp.reshape(loss_act, (1, 1))
    lsact_ref[...] = jnp.reshape(loss_sact, (1, 1))
    logits_act_ref[...] = logits_act
    logits_sact_ref[...] = logits_sact
    logits_role_ref[...] = logits_role


def _tc_heads(h2s, wa, ba, wb, bb, g, be, bv, bact, act_cids, sact_cids,
              actor_cids,
              Wa1, ba1, Wa2, ba2, Ws1, bs1, Ws2, bs2, Wr1, br1, Wr2, br2):
    out_shape = (
        jax.ShapeDtypeStruct((1, 1), jnp.float32),
        jax.ShapeDtypeStruct((1, 1), jnp.float32),
        jax.ShapeDtypeStruct((1, 1), jnp.float32),
        jax.ShapeDtypeStruct((NUM_VIDEOS, NUM_ACT), jnp.float32),
        jax.ShapeDtypeStruct((NUM_VIDEOS, NUM_SACT), jnp.float32),
        jax.ShapeDtypeStruct((NUM_CHUNKS * (SEG - 1), NUM_ACTOR), jnp.float32),
    )
    return pl.pallas_call(_heads_body, out_shape=out_shape)(
        h2s, wa, ba.reshape(1, -1), wb, bb.reshape(1, -1),
        g.reshape(1, -1), be.reshape(1, -1),
        bv.reshape(N, 1), bact.reshape(N, 1),
        act_cids.reshape(-1, 1), sact_cids.reshape(-1, 1), actor_cids.reshape(-1, 1),
        Wa1, ba1.reshape(1, -1), Wa2, ba2.reshape(1, -1),
        Ws1, bs1.reshape(1, -1), Ws2, bs2.reshape(1, -1),
        Wr1, br1.reshape(1, -1), Wr2, br2.reshape(1, -1),
    )


def kernel(x, edge_index, batch_video, batch_actor, act_cids, sact_cids,
           actor_cids,
           W1a, b1a, W1b, b1b, W2a, b2a, W2b, b2b, W3a, b3a, W3b, b3b,
           Wa1, ba1, Wa2, ba2, Ws1, bs1, Ws2, bs2, Wr1, br1, Wr2, br2,
           g1, be1, g2, be2, g3, be3):
    src = edge_index[0]
    dst = edge_index[1]
    nw = NC * NS

    nblk_es = E // nw // K // BLK
    src_es = src.reshape(nw, nblk_es, BLK, K)
    dst_es = dst.reshape(nw, nblk_es, BLK, K)

    nblk_fs = E // NS // K // BLK
    src_fs = jnp.concatenate([src, src + N]).reshape(nw, nblk_fs, BLK, K)
    dst_fs = jnp.concatenate([dst, dst]).reshape(nw, nblk_fs, BLK, K)

    h2 = _make_sc_agg(nblk_es, 0)(x, src_es, dst_es)
    h = _tc_layer(h2, W1a, b1a, W1b, b1b, g1, be1, NUM_FEATS, "es", x=x)

    h2 = _make_sc_agg(nblk_fs, N)(h, src_fs, dst_fs)
    h = _tc_layer(h2, W2a, b2a, W2b, b2b, g2, be2, DIM // 2, "fs")

    h2 = _make_sc_agg(nblk_fs, N)(h, src_fs, dst_fs)

    loss, lact, lsact, logits_act, logits_sact, logits_role = _tc_heads(
        h2, W3a, b3a, W3b, b3b, g3, be3,
        batch_video, batch_actor, act_cids, sact_cids, actor_cids,
        Wa1, ba1, Wa2, ba2, Ws1, bs1, Ws2, bs2, Wr1, br1, Wr2, br2,
    )
    return (
        loss.reshape(()), lact.reshape(()), lsact.reshape(()),
        logits_act, logits_sact, logits_role,
    )


# speedup vs baseline: 9.6462x; 1.0092x over previous
import functools

import jax
import jax.numpy as jnp
from jax import lax
from jax.experimental import pallas as pl
from jax.experimental.pallas import tpu as pltpu
from jax.experimental.pallas import tpu_sc as plsc

N = 10000
E = 320000
NUM_FEATS = 128
DIM = 256
NUM_ACT = 20
NUM_SACT = 91
NUM_ACTOR = 26
NUM_VIDEOS = 16
NUM_CHUNKS = 4
CHUNK = 2500
SEG = 9

NC = 2
NS = 16
SLAB = 640
K = 80


NB = 4


@functools.cache
def _make_sc_agg(nblk: int, blk: int, init_stride: int):
    BLK = blk
    mesh = plsc.VectorSubcoreMesh(
        core_axis_name="c", subcore_axis_name="s", num_cores=NC, num_subcores=NS
    )
    dh = DIM // 2

    scratch = [
        pltpu.VMEM_SHARED((N, dh), jnp.float32),
        pltpu.VMEM((BLK, K), jnp.int32),
        pltpu.VMEM((BLK, K), jnp.int32),
    ]
    scratch += [pltpu.VMEM((K, dh), jnp.float32)] * NB
    scratch += [pltpu.SemaphoreType.DMA] * (2 * NB)

    @functools.partial(
        pl.kernel,
        out_type=jax.ShapeDtypeStruct((2 * N, dh), jnp.float32),
        mesh=mesh,
        scratch_types=scratch,
    )
    def sc_agg(tab_hbm, sidx_hbm, didx_hbm, out_hbm, acc, sidx, didx, *bufs):
        rows = bufs[:NB]
        gsem = bufs[NB:2 * NB]
        ssem = bufs[2 * NB:]
        c = lax.axis_index("c")
        s = lax.axis_index("s")
        w = c * NS + s
        row0 = pl.multiple_of(jnp.minimum(s * SLAB, N - SLAB), 8)

        def fire_gather(k, b):
            pltpu.async_copy(tab_hbm.at[sidx.at[k]], rows[b], gsem[b])

        def wait_gather(k, b):
            pltpu.make_async_copy(tab_hbm.at[sidx.at[k]], rows[b], gsem[b]).wait()

        def fire_scatter(k, b):
            pltpu.async_copy(rows[b], acc.at[didx.at[k]], ssem[b], add=True)

        def wait_scatter(k, b):
            pltpu.make_async_copy(rows[b], acc.at[didx.at[k]], ssem[b]).wait()

        def step(k, b, do_wait=True, do_gather=True):
            tb = (b + 2) % NB
            if do_wait:
                wait_scatter(k - 2, tb)
            if do_gather:
                fire_gather(k + 2, tb)
            wait_gather(k, b)
            fire_scatter(k, b)

        k2_max = (BLK - 6) // NB

        def run_block_steps():
            for k in range(NB):
                step(k, k % NB, do_wait=k >= 2, do_gather=k + 2 < BLK)

            def round_body(k2, carry2):
                k0 = k2 * NB
                for b in range(NB):
                    step(k0 + b, b)
                return carry2

            lax.fori_loop(1, k2_max + 1, round_body, 0, unroll=False)
            for k in range((k2_max + 1) * NB, BLK):
                step(k, k % NB, do_gather=k + 2 < BLK)
            wait_scatter(BLK - 2, (BLK - 2) % NB)
            wait_scatter(BLK - 1, (BLK - 1) % NB)

        pltpu.sync_copy(sidx_hbm.at[w, 0], sidx)
        pltpu.sync_copy(didx_hbm.at[w, 0], didx)
        fire_gather(0, 0)
        fire_gather(1, 1)
        pltpu.sync_copy(
            tab_hbm.at[pl.ds(pl.multiple_of(c * init_stride + row0, 8), SLAB)],
            acc.at[pl.ds(row0, SLAB)],
        )
        plsc.subcore_barrier()
        run_block_steps()

        def block_body(j, carry):
            pltpu.sync_copy(sidx_hbm.at[w, j], sidx)
            pltpu.sync_copy(didx_hbm.at[w, j], didx)
            fire_gather(0, 0)
            fire_gather(1, 1)
            run_block_steps()
            return carry

        lax.fori_loop(1, nblk, block_body, 0, unroll=False)

        plsc.subcore_barrier()
        pltpu.sync_copy(
            acc.at[pl.ds(row0, SLAB)],
            out_hbm.at[pl.ds(pl.multiple_of(c * N + row0, 8), SLAB)],
        )

    return sc_agg


def _layer_body(dh, mode, h2_ref, *refs):
    if mode == "es":
        x_ref, wa_ref, ba_ref, wb_ref, bb_ref, g_ref, be_ref, out_ref = refs
    else:
        wa_ref, ba_ref, wb_ref, bb_ref, g_ref, be_ref, out_ref = refs
    lo = h2_ref[:N, :]
    hi = h2_ref[N:, :]
    if mode == "fs":
        t = (
            jnp.dot(lo, wa_ref[:dh, :], preferred_element_type=jnp.float32)
            + jnp.dot(hi, wa_ref[dh:, :], preferred_element_type=jnp.float32)
            + ba_ref[...]
        )
    else:
        z = lo + hi - x_ref[...]
        t = jnp.dot(z, wa_ref[...], preferred_element_type=jnp.float32) + ba_ref[...]
    t = jnp.maximum(t, 0.0)
    u = jnp.dot(t, wb_ref[...], preferred_element_type=jnp.float32) + bb_ref[...]
    u = jnp.maximum(u, 0.0)
    m = jnp.mean(u, axis=0, keepdims=True)
    v = jnp.mean(u * u, axis=0, keepdims=True) - m * m
    h = (u - m) * (g_ref[...] * jax.lax.rsqrt(v + 1e-5)) + be_ref[...]
    out_ref[:N, :] = h[:, : DIM // 2]
    out_ref[N:, :] = h[:, DIM // 2 :]


def _tc_layer(h2s, wa, ba, wb, bb, g, be, dh, mode, x=None):
    extra = (x,) if mode == "es" else ()
    return pl.pallas_call(
        functools.partial(_layer_body, dh, mode),
        out_shape=jax.ShapeDtypeStruct((2 * N, DIM // 2), jnp.float32),
    )(h2s, *extra, wa, ba.reshape(1, -1), wb, bb.reshape(1, -1),
      g.reshape(1, -1), be.reshape(1, -1))


def _log_softmax(x):
    x = x - jnp.max(x, axis=1, keepdims=True)
    return x - jnp.log(jnp.sum(jnp.exp(x), axis=1, keepdims=True))


def _ce_loss(logits, y_ref, nclass):
    ls = _log_softmax(logits)
    oh = (y_ref[...] == lax.broadcasted_iota(jnp.int32, (y_ref.shape[0], nclass), 1)
          ).astype(jnp.float32)
    return -jnp.sum(oh * ls) / y_ref.shape[0]


def _heads_body(h2_ref, wa_ref, ba_ref, wb_ref, bb_ref, g_ref, be_ref,
                bv_ref, ba2d_ref, act_ref, sact_ref, actor_ref,
                wa1_ref, ba1_ref, wa2_ref, ba2_ref,
                ws1_ref, bs1_ref, ws2_ref, bs2_ref,
                wr1_ref, br1_ref, wr2_ref, br2_ref,
                loss_ref, lact_ref, lsact_ref,
                logits_act_ref, logits_sact_ref, logits_role_ref):
    f32 = jnp.float32
    dh = DIM // 2
    cdot = functools.partial(
        lax.dot_general,
        dimension_numbers=(((0,), (0,)), ((), ())),
        preferred_element_type=f32,
    )
    t = (
        jnp.dot(h2_ref[:N, :], wa_ref[:dh, :], preferred_element_type=f32)
        + jnp.dot(h2_ref[N:, :], wa_ref[dh:, :], preferred_element_type=f32)
        + ba_ref[...]
    )
    t = jnp.maximum(t, 0.0)
    u = jnp.dot(t, wb_ref[...], preferred_element_type=f32) + bb_ref[...]
    u = jnp.maximum(u, 0.0)
    m = jnp.mean(u, axis=0, keepdims=True)
    v = jnp.mean(u * u, axis=0, keepdims=True) - m * m
    h = (u - m) * (g_ref[...] * jax.lax.rsqrt(v + 1e-5)) + be_ref[...]
    h_lo = h[:, :dh]
    h_hi = h[:, dh:]
    ba_ref = ba2d_ref

    ohv = (bv_ref[...] == lax.broadcasted_iota(jnp.int32, (N, NUM_VIDEOS), 1)
           ).astype(f32)
    cnt = jnp.sum(ohv, axis=0, keepdims=True)
    inv = 1.0 / jnp.maximum(cnt, 1.0)
    pv = jnp.concatenate([cdot(ohv, h_lo), cdot(ohv, h_hi)], axis=1) * inv.T

    logits_act = (
        jnp.dot(jnp.maximum(jnp.dot(pv, wa1_ref[...], preferred_element_type=f32)
                            + ba1_ref[...], 0.0),
                wa2_ref[...], preferred_element_type=f32) + ba2_ref[...]
    )
    logits_sact = (
        jnp.dot(jnp.maximum(jnp.dot(pv, ws1_ref[...], preferred_element_type=f32)
                            + bs1_ref[...], 0.0),
                ws2_ref[...], preferred_element_type=f32) + bs2_ref[...]
    )

    embs = []
    for i in range(NUM_CHUNKS):
        sb = ba_ref[i * CHUNK:(i + 1) * CHUNK, :]
        oha = (sb == lax.broadcasted_iota(jnp.int32, (CHUNK, SEG), 1)).astype(f32)
        ccnt = jnp.sum(oha, axis=0, keepdims=True)
        cinv = 1.0 / jnp.maximum(ccnt, 1.0)
        s_lo = cdot(oha, h_lo[i * CHUNK:(i + 1) * CHUNK, :])
        s_hi = cdot(oha, h_hi[i * CHUNK:(i + 1) * CHUNK, :])
        emb = jnp.concatenate([s_lo, s_hi], axis=1) * cinv.T
        embs.append(emb[1:, :])
    emb_act = jnp.concatenate(embs, axis=0)

    logits_role = (
        jnp.dot(jnp.maximum(jnp.dot(emb_act, wr1_ref[...], preferred_element_type=f32)
                            + br1_ref[...], 0.0),
                wr2_ref[...], preferred_element_type=f32) + br2_ref[...]
    )

    loss_act = _ce_loss(logits_act, act_ref, NUM_ACT)
    loss_sact = _ce_loss(logits_sact, sact_ref, NUM_SACT)
    loss_role = _ce_loss(logits_role, actor_ref, NUM_ACTOR)

    loss_ref[...] = jnp.reshape(loss_role, (1, 1))
    lact_ref[...] = jnp.reshape(loss_act, (1, 1))
    lsact_ref[...] = jnp.reshape(loss_sact, (1, 1))
    logits_act_ref[...] = logits_act
    logits_sact_ref[...] = logits_sact
    logits_role_ref[...] = logits_role


def _tc_heads(h2s, wa, ba, wb, bb, g, be, bv, bact, act_cids, sact_cids,
              actor_cids,
              Wa1, ba1, Wa2, ba2, Ws1, bs1, Ws2, bs2, Wr1, br1, Wr2, br2):
    out_shape = (
        jax.ShapeDtypeStruct((1, 1), jnp.float32),
        jax.ShapeDtypeStruct((1, 1), jnp.float32),
        jax.ShapeDtypeStruct((1, 1), jnp.float32),
        jax.ShapeDtypeStruct((NUM_VIDEOS, NUM_ACT), jnp.float32),
        jax.ShapeDtypeStruct((NUM_VIDEOS, NUM_SACT), jnp.float32),
        jax.ShapeDtypeStruct((NUM_CHUNKS * (SEG - 1), NUM_ACTOR), jnp.float32),
    )
    return pl.pallas_call(_heads_body, out_shape=out_shape)(
        h2s, wa, ba.reshape(1, -1), wb, bb.reshape(1, -1),
        g.reshape(1, -1), be.reshape(1, -1),
        bv.reshape(N, 1), bact.reshape(N, 1),
        act_cids.reshape(-1, 1), sact_cids.reshape(-1, 1), actor_cids.reshape(-1, 1),
        Wa1, ba1.reshape(1, -1), Wa2, ba2.reshape(1, -1),
        Ws1, bs1.reshape(1, -1), Ws2, bs2.reshape(1, -1),
        Wr1, br1.reshape(1, -1), Wr2, br2.reshape(1, -1),
    )


def kernel(x, edge_index, batch_video, batch_actor, act_cids, sact_cids,
           actor_cids,
           W1a, b1a, W1b, b1b, W2a, b2a, W2b, b2b, W3a, b3a, W3b, b3b,
           Wa1, ba1, Wa2, ba2, Ws1, bs1, Ws2, bs2, Wr1, br1, Wr2, br2,
           g1, be1, g2, be2, g3, be3):
    src = edge_index[0]
    dst = edge_index[1]
    nw = NC * NS

    blk_es = 25
    nblk_es = E // nw // K // blk_es
    src_es = src.reshape(nw, nblk_es, blk_es, K)
    dst_es = dst.reshape(nw, nblk_es, blk_es, K)

    blk_fs = 25
    nblk_fs = E // NS // K // blk_fs
    src_fs = jnp.concatenate([src, src + N]).reshape(nw, nblk_fs, blk_fs, K)
    dst_fs = jnp.concatenate([dst, dst]).reshape(nw, nblk_fs, blk_fs, K)

    h2 = _make_sc_agg(nblk_es, blk_es, 0)(x, src_es, dst_es)
    h = _tc_layer(h2, W1a, b1a, W1b, b1b, g1, be1, NUM_FEATS, "es", x=x)

    h2 = _make_sc_agg(nblk_fs, blk_fs, N)(h, src_fs, dst_fs)
    h = _tc_layer(h2, W2a, b2a, W2b, b2b, g2, be2, DIM // 2, "fs")

    h2 = _make_sc_agg(nblk_fs, blk_fs, N)(h, src_fs, dst_fs)

    loss, lact, lsact, logits_act, logits_sact, logits_role = _tc_heads(
        h2, W3a, b3a, W3b, b3b, g3, be3,
        batch_video, batch_actor, act_cids, sact_cids, actor_cids,
        Wa1, ba1, Wa2, ba2, Ws1, bs1, Ws2, bs2, Wr1, br1, Wr2, br2,
    )
    return (
        loss.reshape(()), lact.reshape(()), lsact.reshape(()),
        logits_act, logits_sact, logits_role,
    )
